# baseline (speedup 1.0000x reference)
import functools

import jax
import jax.numpy as jnp
from jax import lax
from jax.experimental import pallas as pl
from jax.experimental.pallas import tpu as pltpu
from jax.experimental.pallas import tpu_sc as plsc

N = 10000
E = 320000
D = 128
L = 4
ALPHA = 0.5
EPS = 1e-5

NW = 32
CHUNK = 128
NBUF = 2
EPAD = 327680
TOTCHUNKS = EPAD // CHUNK
NCH0 = 80
NCH1 = (TOTCHUNKS - 16 * NCH0) // 16
NCHUNKS = TOTCHUNKS // NW
NPAD = 10112
ROWS_PER_TILE = NPAD // 16

_sc_mesh = plsc.VectorSubcoreMesh(core_axis_name="c", subcore_axis_name="s")


@functools.partial(
    pl.kernel,
    out_type=jax.ShapeDtypeStruct((2, NPAD, D), jnp.float32),
    mesh=_sc_mesh,
    scratch_types=[
        pltpu.VMEM((NBUF, 2, CHUNK), jnp.int32),
        pltpu.VMEM((NBUF, CHUNK, D), jnp.float32),
        pltpu.VMEM((8, D), jnp.float32),
        pltpu.VMEM_SHARED((NPAD, D), jnp.float32),
    ] + [pltpu.SemaphoreType.DMA] * (2 * NBUF),
)
def _sc_edge_pass(dsup_hbm, idx_hbm, zeros_hbm, out_hbm,
                  idx_v, buf_v, zero_v, agg_sh, *sems):
    isem = sems[:NBUF]
    gsem = sems[NBUF:]
    c = lax.axis_index("c")
    s = lax.axis_index("s")
    start = jnp.where(c == 0, s * NCH0, 16 * NCH0 + s * NCH1)
    nch = jnp.where(c == 0, NCH0, NCH1)

    pltpu.sync_copy(zeros_hbm, zero_v)

    base = s * ROWS_PER_TILE
    def _zero(k, carry):
        pltpu.sync_copy(zero_v, agg_sh.at[pl.ds(base + k * 8, 8)])
        return carry
    lax.fori_loop(0, ROWS_PER_TILE // 8, _zero, 0)
    plsc.subcore_barrier()

    pltpu.sync_copy(idx_hbm.at[start], idx_v.at[0])
    pltpu.async_copy(dsup_hbm.at[idx_v.at[0, 0]], buf_v.at[0], gsem[0])
    pltpu.async_copy(idx_hbm.at[start + 1], idx_v.at[1], isem[1])

    def _step(j, carry):
        for u in range(NBUF):
            jj = j * NBUF + u
            b = u
            nb = 1 - u

            @pl.when(jj + 1 < nch)
            def _():
                pltpu.make_async_copy(idx_hbm.at[start + jj + 1],
                                      idx_v.at[nb], isem[nb]).wait()
                pltpu.async_copy(dsup_hbm.at[idx_v.at[nb, 0]], buf_v.at[nb],
                                 gsem[nb])
            pltpu.make_async_copy(dsup_hbm.at[idx_v.at[b, 0]], buf_v.at[b],
                                  gsem[b]).wait()
            pltpu.sync_copy(buf_v.at[b], agg_sh.at[idx_v.at[b, 1]], add=True)

            @pl.when(jj + 2 < nch)
            def _():
                pltpu.async_copy(idx_hbm.at[start + jj + 2], idx_v.at[b],
                                 isem[b])
        return carry
    lax.fori_loop(0, nch // NBUF, _step, 0)

    plsc.subcore_barrier()
    pltpu.sync_copy(agg_sh.at[pl.ds(base, ROWS_PER_TILE)],
                    out_hbm.at[c, pl.ds(base, ROWS_PER_TILE)])


@functools.partial(
    pl.kernel,
    out_type=jax.ShapeDtypeStruct((2, NPAD, 16), jnp.float32),
    mesh=_sc_mesh,
    scratch_types=[
        pltpu.VMEM((NCHUNKS, 2, CHUNK), jnp.int32),
        pltpu.VMEM((CHUNK, 16), jnp.float32),
        pltpu.VMEM((8, 16), jnp.float32),
        pltpu.VMEM_SHARED((NPAD, 16), jnp.float32),
    ],
)
def _sc_degree(idx_hbm, ones_hbm, zeros_hbm, out_hbm,
               idx_v, ones_v, zero_v, deg_sh):
    c = lax.axis_index("c")
    s = lax.axis_index("s")
    wid = c * 16 + s

    pltpu.sync_copy(idx_hbm.at[wid], idx_v)
    pltpu.sync_copy(ones_hbm, ones_v)
    pltpu.sync_copy(zeros_hbm, zero_v)

    base = s * ROWS_PER_TILE
    def _zero(k, carry):
        pltpu.sync_copy(zero_v, deg_sh.at[pl.ds(base + k * 8, 8)])
        return carry
    lax.fori_loop(0, ROWS_PER_TILE // 8, _zero, 0)
    plsc.subcore_barrier()

    def _step(j, carry):
        pltpu.sync_copy(ones_v, deg_sh.at[idx_v.at[j, 1]], add=True)
        return carry
    lax.fori_loop(0, NCHUNKS, _step, 0)

    plsc.subcore_barrier()
    pltpu.sync_copy(deg_sh.at[pl.ds(base, ROWS_PER_TILE)],
                    out_hbm.at[c, pl.ds(base, ROWS_PER_TILE)])


def _pre(x, w1, dinv):
    sup = x + jnp.dot(x, w1, preferred_element_type=jnp.float32)
    return sup, dinv * sup


def _post(aggp_ref, sup, init, dinv, g, b, prev):
    out = (dinv * (aggp_ref[0, 0:N, :] + aggp_ref[1, 0:N, :])
           + (dinv * dinv) * sup + init)
    m = jnp.mean(out, axis=0, keepdims=True)
    v = jnp.mean((out - m) * (out - m), axis=0, keepdims=True)
    outn = g * (out - m) * lax.rsqrt(v + EPS) + b
    return jax.nn.relu(outn) + prev


def _initial(h0, w2):
    return ALPHA * h0 + jnp.dot(h0, w2, preferred_element_type=jnp.float32)


def _start_body(x_ref, w0_ref, b0_ref, degp_ref, w10_ref,
                h_ref, dinv_ref, sup_ref, dsup_ref):
    h = jax.nn.relu(
        jnp.dot(x_ref[...], w0_ref[...], preferred_element_type=jnp.float32)
        + b0_ref[...])
    h_ref[...] = h
    deg = degp_ref[0, 0:N, 0:1] + degp_ref[1, 0:N, 0:1] + 1.0
    dinv = lax.rsqrt(deg)
    dinv_ref[...] = dinv
    sup, dsup = _pre(h, w10_ref[...], dinv)
    sup_ref[...] = sup
    dsup_ref[...] = dsup


def _mid_body(aggp_ref, sup_ref, h0_ref, w2_ref, dinv_ref, g_ref, b_ref,
              prev_ref, w1_ref, h_ref, sup2_ref, dsup_ref):
    dinv = dinv_ref[...]
    init = _initial(h0_ref[...], w2_ref[...])
    h = _post(aggp_ref, sup_ref[...], init, dinv, g_ref[...],
              b_ref[...], prev_ref[...])
    h_ref[...] = h
    sup, dsup = _pre(h, w1_ref[...], dinv)
    sup2_ref[...] = sup
    dsup_ref[...] = dsup


def _end_body(aggp_ref, sup_ref, h0_ref, w2_ref, dinv_ref, g_ref, b_ref,
              prev_ref, wl_ref, bl_ref, o_ref):
    init = _initial(h0_ref[...], w2_ref[...])
    h = _post(aggp_ref, sup_ref[...], init, dinv_ref[...],
              g_ref[...], b_ref[...], prev_ref[...])
    o_ref[...] = (jnp.dot(h, wl_ref[...], preferred_element_type=jnp.float32)
                  + bl_ref[...])


def kernel(x, edge_index, W0, b0, W1, W2, gamma, beta, Wl, bl):
    f32 = jnp.float32
    row = edge_index[0].astype(jnp.int32)
    col = edge_index[1].astype(jnp.int32)
    pad = EPAD - E
    pad_col = N + jnp.arange(pad, dtype=jnp.int32) % (NPAD - N)
    row_p = jnp.concatenate([row, jnp.zeros((pad,), jnp.int32)])
    col_p = jnp.concatenate([col, pad_col])
    idx = jnp.stack([row_p.reshape(NW, NCHUNKS, CHUNK),
                     col_p.reshape(NW, NCHUNKS, CHUNK)], axis=2)
    idx_flat = idx.reshape(TOTCHUNKS, 2, CHUNK)

    zeros16 = jnp.zeros((8, 16), f32)
    ones16 = jnp.ones((CHUNK, 16), f32)
    zeros128 = jnp.zeros((8, D), f32)

    degp = _sc_degree(idx, ones16, zeros16)

    nd = jax.ShapeDtypeStruct((N, D), f32)
    h0, dinv, sup, dsup = pl.pallas_call(
        _start_body,
        out_shape=(nd, jax.ShapeDtypeStruct((N, 1), f32), nd, nd),
    )(x, W0, b0.reshape(1, D), degp, W1[0])

    prev = h0
    for i in range(L - 1):
        aggp = _sc_edge_pass(dsup, idx_flat, zeros128)
        prev, sup, dsup = pl.pallas_call(
            _mid_body, out_shape=(nd, nd, nd),
        )(aggp, sup, h0, W2[i], dinv, gamma[i].reshape(1, D),
          beta[i].reshape(1, D), prev, W1[i + 1])

    aggp = _sc_edge_pass(dsup, idx_flat, zeros128)
    return pl.pallas_call(
        _end_body, out_shape=nd,
    )(aggp, sup, h0, W2[L - 1], dinv, gamma[L - 1].reshape(1, D),
      beta[L - 1].reshape(1, D), prev, Wl, bl.reshape(1, D))


# speedup vs baseline: 1.2200x; 1.2200x over previous
import functools

import jax
import jax.numpy as jnp
from jax import lax
from jax.experimental import pallas as pl
from jax.experimental.pallas import tpu as pltpu
from jax.experimental.pallas import tpu_sc as plsc

N = 10000
E = 320000
D = 128
L = 4
ALPHA = 0.5
EPS = 1e-5

NW = 32
CHUNK = 128
NBUF = 2
EPAD = 327680
TOTCHUNKS = EPAD // CHUNK
NCH0 = 80
NCH1 = (TOTCHUNKS - 16 * NCH0) // 16
NCHUNKS = TOTCHUNKS // NW
NPAD = 10112
ROWS_PER_TILE = NPAD // 16

_sc_mesh = plsc.VectorSubcoreMesh(core_axis_name="c", subcore_axis_name="s")


@functools.partial(
    pl.kernel,
    out_type=jax.ShapeDtypeStruct((2, NPAD, D), jnp.float32),
    mesh=_sc_mesh,
    scratch_types=[
        pltpu.VMEM((NBUF, 2, CHUNK), jnp.int32),
        pltpu.VMEM((NBUF, CHUNK, D), jnp.float32),
        pltpu.VMEM((8, D), jnp.float32),
        pltpu.VMEM_SHARED((NPAD, D), jnp.float32),
    ] + [pltpu.SemaphoreType.DMA] * (2 * NBUF),
)
def _sc_edge_pass(dsup_hbm, idx_hbm, zeros_hbm, out_hbm,
                  idx_v, buf_v, zero_v, agg_sh, *sems):
    isem = sems[:NBUF]
    gsem = sems[NBUF:]
    c = lax.axis_index("c")
    s = lax.axis_index("s")
    start = jnp.where(c == 0, s * NCH0, 16 * NCH0 + s * NCH1)
    nch = jnp.where(c == 0, NCH0, NCH1)

    pltpu.sync_copy(zeros_hbm, zero_v)

    base = s * ROWS_PER_TILE
    def _zero(k, carry):
        pltpu.sync_copy(zero_v, agg_sh.at[pl.ds(base + k * 8, 8)])
        return carry
    lax.fori_loop(0, ROWS_PER_TILE // 8, _zero, 0)
    plsc.subcore_barrier()

    pltpu.sync_copy(idx_hbm.at[start], idx_v.at[0])
    pltpu.async_copy(dsup_hbm.at[idx_v.at[0, 0]], buf_v.at[0], gsem[0])
    pltpu.async_copy(idx_hbm.at[start + 1], idx_v.at[1], isem[1])

    def _step(j, carry):
        for u in range(NBUF):
            jj = j * NBUF + u
            b = u
            nb = 1 - u

            @pl.when(jj + 1 < nch)
            def _():
                pltpu.make_async_copy(idx_hbm.at[start + jj + 1],
                                      idx_v.at[nb], isem[nb]).wait()
                pltpu.async_copy(dsup_hbm.at[idx_v.at[nb, 0]], buf_v.at[nb],
                                 gsem[nb])
            pltpu.make_async_copy(dsup_hbm.at[idx_v.at[b, 0]], buf_v.at[b],
                                  gsem[b]).wait()
            pltpu.sync_copy(buf_v.at[b], agg_sh.at[idx_v.at[b, 1]], add=True)

            @pl.when(jj + 2 < nch)
            def _():
                pltpu.async_copy(idx_hbm.at[start + jj + 2], idx_v.at[b],
                                 isem[b])
        return carry
    lax.fori_loop(0, nch // NBUF, _step, 0)

    plsc.subcore_barrier()
    pltpu.sync_copy(agg_sh.at[pl.ds(base, ROWS_PER_TILE)],
                    out_hbm.at[c, pl.ds(base, ROWS_PER_TILE)])


@functools.partial(
    pl.kernel,
    out_type=jax.ShapeDtypeStruct((2, NPAD, 16), jnp.float32),
    mesh=_sc_mesh,
    scratch_types=[
        pltpu.VMEM((NCHUNKS, 2, CHUNK), jnp.int32),
        pltpu.VMEM((CHUNK, 16), jnp.float32),
        pltpu.VMEM((8, 16), jnp.float32),
        pltpu.VMEM_SHARED((NPAD, 16), jnp.float32),
    ],
)
def _sc_degree(idx_hbm, ones_hbm, zeros_hbm, out_hbm,
               idx_v, ones_v, zero_v, deg_sh):
    c = lax.axis_index("c")
    s = lax.axis_index("s")
    wid = c * 16 + s

    pltpu.sync_copy(idx_hbm.at[wid], idx_v)
    pltpu.sync_copy(ones_hbm, ones_v)
    pltpu.sync_copy(zeros_hbm, zero_v)

    base = s * ROWS_PER_TILE
    def _zero(k, carry):
        pltpu.sync_copy(zero_v, deg_sh.at[pl.ds(base + k * 8, 8)])
        return carry
    lax.fori_loop(0, ROWS_PER_TILE // 8, _zero, 0)
    plsc.subcore_barrier()

    def _step(j, carry):
        pltpu.sync_copy(ones_v, deg_sh.at[idx_v.at[j, 1]], add=True)
        return carry
    lax.fori_loop(0, NCHUNKS, _step, 0)

    plsc.subcore_barrier()
    pltpu.sync_copy(deg_sh.at[pl.ds(base, ROWS_PER_TILE)],
                    out_hbm.at[c, pl.ds(base, ROWS_PER_TILE)])


def _h_body(x_ref, w_ref, b_ref, o_ref):
    o_ref[...] = jax.nn.relu(
        jnp.dot(x_ref[...], w_ref[...], preferred_element_type=jnp.float32)
        + b_ref[...])


def _prep_body(degp_ref, h_ref, w2_ref, dinv_ref, init_ref):
    deg = degp_ref[0, 0:N, 0:1] + degp_ref[1, 0:N, 0:1] + 1.0
    dinv = lax.rsqrt(deg)
    dinv_ref[...] = dinv
    h = h_ref[...]
    for l in range(L):
        init_ref[l] = ALPHA * h + jnp.dot(h, w2_ref[l],
                                          preferred_element_type=jnp.float32)


def _pre_body(x_ref, w_ref, dinv_ref, sup_ref, dsup_ref):
    x = x_ref[...]
    sup = x + jnp.dot(x, w_ref[...], preferred_element_type=jnp.float32)
    sup_ref[...] = sup
    dsup_ref[...] = dinv_ref[...] * sup


def _post_body(aggp_ref, sup_ref, init_ref, dinv_ref, g_ref, b_ref, prev_ref,
               h_ref):
    dinv = dinv_ref[...]
    out = (dinv * (aggp_ref[0, 0:N, :] + aggp_ref[1, 0:N, :])
           + (dinv * dinv) * sup_ref[...] + init_ref[...])
    m = jnp.mean(out, axis=0, keepdims=True)
    v = jnp.mean((out - m) * (out - m), axis=0, keepdims=True)
    outn = g_ref[...] * (out - m) * lax.rsqrt(v + EPS) + b_ref[...]
    h_ref[...] = jax.nn.relu(outn) + prev_ref[...]


def _final_body(x_ref, w_ref, b_ref, o_ref):
    o_ref[...] = (jnp.dot(x_ref[...], w_ref[...],
                          preferred_element_type=jnp.float32) + b_ref[...])


def kernel(x, edge_index, W0, b0, W1, W2, gamma, beta, Wl, bl):
    f32 = jnp.float32
    row = edge_index[0].astype(jnp.int32)
    col = edge_index[1].astype(jnp.int32)
    pad = EPAD - E
    pad_col = N + jnp.arange(pad, dtype=jnp.int32) % (NPAD - N)
    row_p = jnp.concatenate([row, jnp.zeros((pad,), jnp.int32)])
    col_p = jnp.concatenate([col, pad_col])
    idx = jnp.stack([row_p.reshape(NW, NCHUNKS, CHUNK),
                     col_p.reshape(NW, NCHUNKS, CHUNK)], axis=2)
    idx_flat = idx.reshape(TOTCHUNKS, 2, CHUNK)

    zeros16 = jnp.zeros((8, 16), f32)
    ones16 = jnp.ones((CHUNK, 16), f32)
    zeros128 = jnp.zeros((8, D), f32)

    degp = _sc_degree(idx, ones16, zeros16)

    nd = jax.ShapeDtypeStruct((N, D), f32)
    h = pl.pallas_call(_h_body, out_shape=nd)(x, W0, b0.reshape(1, D))

    dinv, init_all = pl.pallas_call(
        _prep_body,
        out_shape=(jax.ShapeDtypeStruct((N, 1), f32),
                   jax.ShapeDtypeStruct((L, N, D), f32)),
    )(degp, h, W2)

    prev = h
    xcur = h
    for i in range(L):
        sup, dsup = pl.pallas_call(
            _pre_body, out_shape=(nd, nd),
        )(xcur, W1[i], dinv)

        aggp = _sc_edge_pass(dsup, idx_flat, zeros128)

        hnew = pl.pallas_call(
            _post_body, out_shape=nd,
        )(aggp, sup, init_all[i], dinv, gamma[i].reshape(1, D),
          beta[i].reshape(1, D), prev)
        prev = hnew
        xcur = hnew

    return pl.pallas_call(_final_body, out_shape=nd)(xcur, Wl,
                                                     bl.reshape(1, D))
